# baseline (speedup 1.0000x reference)
import jax
import jax.numpy as jnp
from jax.experimental import pallas as pl

_NUM_SELECT = 300


def _sigmoid_kernel(x_ref, o_ref):
    o_ref[...] = jax.nn.sigmoid(x_ref[...])


def kernel(pred_logits, pred_boxes, target_sizes):
    B, Q, C = pred_logits.shape
    prob = pl.pallas_call(
        _sigmoid_kernel,
        out_shape=jax.ShapeDtypeStruct((B, Q, C), jnp.float32),
        grid=(B,),
        in_specs=[pl.BlockSpec((1, Q, C), lambda b: (b, 0, 0))],
        out_specs=pl.BlockSpec((1, Q, C), lambda b: (b, 0, 0)),
    )(pred_logits)
    topk_values, topk_indexes = jax.lax.top_k(prob.reshape(B, -1), _NUM_SELECT)
    scores = topk_values
    topk_boxes = topk_indexes // C
    labels = topk_indexes % C
    cx, cy, w, h = (pred_boxes[..., i] for i in range(4))
    boxes = jnp.stack([cx - 0.5 * w, cy - 0.5 * h, cx + 0.5 * w, cy + 0.5 * h], axis=-1)
    boxes = jnp.take_along_axis(boxes, topk_boxes[:, :, None], axis=1)
    img_h = target_sizes[:, 0]
    img_w = target_sizes[:, 1]
    scale_fct = jnp.stack([img_w, img_h, img_w, img_h], axis=1).astype(boxes.dtype)
    boxes = boxes * scale_fct[:, None, :]
    return scores, labels, boxes


# speedup vs baseline: 4.1208x; 4.1208x over previous
import functools

import jax
import jax.numpy as jnp
from jax.experimental import pallas as pl
from jax.experimental.pallas import tpu as pltpu

_NUM_SELECT = 300
_BIG = 2**30


def _post_kernel(x_ref, box_ref, scores_ref, labels_ref, boxes_ref, p_ref, m_ref,
                 *, Q, C, n_chunks, n_select):
    for i in range(n_chunks):
        lo = i * 128
        hi = min(lo + 128, Q)
        rows = hi - lo
        p_ref[i, 0:rows, :] = jax.nn.sigmoid(x_ref[0, lo:hi, :])
        if rows < 128:
            p_ref[i, rows:128, :] = jnp.full((128 - rows, C), -1.0, jnp.float32)

    m_ref[...] = jnp.max(p_ref[...], axis=2)

    iota_flat = jax.lax.broadcasted_iota(jnp.int32, (n_chunks, 128), 0) * 128 + \
        jax.lax.broadcasted_iota(jnp.int32, (n_chunks, 128), 1)
    iota_c = jax.lax.broadcasted_iota(jnp.int32, (1, 1, C), 2)
    iota4 = jax.lax.broadcasted_iota(jnp.int32, (1, 4), 1)

    def body(it, carry):
        mv = m_ref[...]
        m = jnp.max(mv)
        q = jnp.min(jnp.where(mv == m, iota_flat, _BIG))
        ci = q // 128
        sj = q - ci * 128
        row = p_ref[pl.ds(ci, 1), pl.ds(sj, 1), :]
        c = jnp.min(jnp.where(row == m, iota_c, _BIG))
        p_ref[pl.ds(ci, 1), pl.ds(sj, 1), :] = jnp.where(iota_c == c, -1.0, row)
        m2 = jnp.max(jnp.where(iota_c == c, -1.0, row))
        m_ref[...] = jnp.where(iota_flat == q, m2, mv)

        scores_ref[0, pl.ds(it, 1), :] = jnp.full((1, 1), m, jnp.float32)
        labels_ref[0, pl.ds(it, 1), :] = jnp.full((1, 1), c, jnp.int32)

        b = box_ref[0, pl.ds(q, 1), :]
        cx = jnp.sum(jnp.where(iota4 == 0, b, 0.0))
        cy = jnp.sum(jnp.where(iota4 == 1, b, 0.0))
        w = jnp.sum(jnp.where(iota4 == 2, b, 0.0))
        h = jnp.sum(jnp.where(iota4 == 3, b, 0.0))
        xyxy = jnp.where(iota4 == 0, cx - 0.5 * w,
                         jnp.where(iota4 == 1, cy - 0.5 * h,
                                   jnp.where(iota4 == 2, cx + 0.5 * w, cy + 0.5 * h)))
        boxes_ref[0, pl.ds(it, 1), :] = xyxy
        return carry

    jax.lax.fori_loop(0, n_select, body, 0)


def kernel(pred_logits, pred_boxes, target_sizes):
    B, Q, C = pred_logits.shape
    n_chunks = (Q + 127) // 128
    k = _NUM_SELECT
    k_pad = ((k + 7) // 8) * 8

    grid = (B,)
    kfun = functools.partial(_post_kernel, Q=Q, C=C, n_chunks=n_chunks, n_select=k)
    scores3, labels3, boxes3 = pl.pallas_call(
        kfun,
        grid=grid,
        in_specs=[
            pl.BlockSpec((1, Q, C), lambda b: (b, 0, 0)),
            pl.BlockSpec((1, Q, 4), lambda b: (b, 0, 0)),
        ],
        out_specs=[
            pl.BlockSpec((1, k_pad, 1), lambda b: (b, 0, 0)),
            pl.BlockSpec((1, k_pad, 1), lambda b: (b, 0, 0)),
            pl.BlockSpec((1, k_pad, 4), lambda b: (b, 0, 0)),
        ],
        out_shape=[
            jax.ShapeDtypeStruct((B, k_pad, 1), jnp.float32),
            jax.ShapeDtypeStruct((B, k_pad, 1), jnp.int32),
            jax.ShapeDtypeStruct((B, k_pad, 4), jnp.float32),
        ],
        scratch_shapes=[
            pltpu.VMEM((n_chunks, 128, C), jnp.float32),
            pltpu.VMEM((n_chunks, 128), jnp.float32),
        ],
    )(pred_logits, pred_boxes)

    scores = scores3[:, :k, 0]
    labels = labels3[:, :k, 0]
    img_h = target_sizes[:, 0]
    img_w = target_sizes[:, 1]
    scale_fct = jnp.stack([img_w, img_h, img_w, img_h], axis=1).astype(jnp.float32)
    boxes = boxes3[:, :k, :] * scale_fct[:, None, :]
    return scores, labels, boxes


# speedup vs baseline: 37.4856x; 9.0968x over previous
import functools

import jax
import jax.numpy as jnp
from jax import lax
from jax.experimental import pallas as pl
from jax.experimental.pallas import tpu as pltpu
from jax.experimental.pallas import tpu_sc as plsc

_NUM_SELECT = 300
_BIG = 2**30
_DEPTH = 6
_NEG = -2.0


def _lists_kernel(x_ref, v_ref, c_ref, p_ref, *, Q, C, n_chunks):
    for i in range(n_chunks):
        lo = i * 128
        hi = min(lo + 128, Q)
        rows = hi - lo
        p_ref[i, 0:rows, :] = jax.nn.sigmoid(x_ref[0, lo:hi, :])
        if rows < 128:
            p_ref[i, rows:128, :] = jnp.full((128 - rows, C), -1.0, jnp.float32)

    iota_c3 = jax.lax.broadcasted_iota(jnp.int32, (n_chunks, 128, C), 2)
    for t in range(_DEPTH):
        p = p_ref[...]
        vals = jnp.max(p, axis=2)
        ct = jnp.min(jnp.where(p == vals[..., None], iota_c3, _BIG), axis=2)
        v_ref[0, t] = vals
        c_ref[0, t] = ct
        if t + 1 < _DEPTH:
            p_ref[...] = jnp.where(iota_c3 == ct[..., None], -1.0, p)


def _sc_extract(v_hbm, c_hbm, boxes_hbm, scores_hbm, labels_hbm, boxout_hbm,
                flag_hbm, vv, cv, bx_ref, m_ref, m2_ref, k_ref, sc_ref, la_ref,
                bi_ref, gb_ref, fl_ref, *, Q, n_rows, k_pad, n_select):
    b = lax.axis_index("s") * 2 + lax.axis_index("c")
    nl = _DEPTH * n_rows
    pltpu.sync_copy(v_hbm.at[pl.ds(b * nl, nl)], vv)
    pltpu.sync_copy(c_hbm.at[pl.ds(b * nl, nl)], cv)
    pltpu.sync_copy(boxes_hbm.at[pl.ds(b * Q * 4, Q * 4)], bx_ref)
    iota = lax.broadcasted_iota(jnp.int32, (16,), 0)
    n_vec = n_rows // 16
    n_vec2 = n_vec // 16

    def init_m(g, _):
        sl = pl.ds(pl.multiple_of(g * 16, 16), 16)
        vvec = vv[sl]
        m_ref[sl] = vvec
        k_ref[sl] = jnp.zeros((16,), jnp.int32)
        base = (g // 16) * 16
        sl2 = pl.ds(pl.multiple_of(base, 16), 16)
        m2_ref[sl2] = jnp.where(iota == g - base, jnp.max(vvec), m2_ref[sl2])
        return 0

    lax.fori_loop(0, n_vec, init_m, 0)

    def init_out(j, _):
        sl = pl.ds(pl.multiple_of(j * 16, 16), 16)
        sc_ref[sl] = jnp.zeros((16,), jnp.float32)
        la_ref[sl] = jnp.zeros((16,), jnp.int32)
        bi_ref[sl] = jnp.zeros((16,), jnp.int32)
        return 0

    lax.fori_loop(0, k_pad // 16, init_out, 0)

    def step(it, flag):
        def mx(i, acc):
            return jnp.maximum(acc, jnp.max(m2_ref[pl.ds(pl.multiple_of(i * 16, 16), 16)]))
        m = lax.fori_loop(0, n_vec2, mx, jnp.float32(-3.0))

        def fe(i, acc):
            v2 = m2_ref[pl.ds(pl.multiple_of(i * 16, 16), 16)]
            return jnp.minimum(acc, jnp.min(jnp.where(v2 == m, iota + i * 16, _BIG)))
        g = lax.fori_loop(0, n_vec2, fe, jnp.int32(_BIG))

        gs = pl.multiple_of(g * 16, 16)
        mv = m_ref[pl.ds(gs, 16)]
        q = jnp.min(jnp.where(mv == m, iota + g * 16, _BIG))
        lane_q = q - g * 16
        kvec = k_ref[pl.ds(gs, 16)]
        kq = jnp.max(jnp.where(iota == lane_q, kvec, 0))
        ovf = kq + 1 >= _DEPTH
        t2 = jnp.minimum(kq + 1, _DEPTH - 1)

        cvec = cv[pl.ds(pl.multiple_of(kq * n_rows + gs, 16), 16)]
        cval = jnp.max(jnp.where(iota == lane_q, cvec, 0))
        nvec = vv[pl.ds(pl.multiple_of(t2 * n_rows + gs, 16), 16)]
        nval = jnp.max(jnp.where(iota == lane_q, nvec, _NEG))
        nval = jnp.where(ovf, jnp.float32(_NEG), nval)

        mv2 = jnp.where(iota == lane_q, nval, mv)
        m_ref[pl.ds(gs, 16)] = mv2
        base = (g // 16) * 16
        sl2 = pl.ds(pl.multiple_of(base, 16), 16)
        m2_ref[sl2] = jnp.where(iota == g - base, jnp.max(mv2), m2_ref[sl2])
        k_ref[pl.ds(gs, 16)] = jnp.where(iota == lane_q, kq + 1, kvec)

        ob = (it // 16) * 16
        slo = pl.ds(pl.multiple_of(ob, 16), 16)
        lane_it = it - ob
        sc_ref[slo] = jnp.where(iota == lane_it, m, sc_ref[slo])
        la_ref[slo] = jnp.where(iota == lane_it, cval, la_ref[slo])
        bi_ref[slo] = jnp.where(iota == lane_it, q, bi_ref[slo])
        return flag | ovf.astype(jnp.int32)

    flag = lax.fori_loop(0, n_select, step, jnp.int32(0))

    def gbvec(v, _):
        qv = plsc.load_gather(bi_ref, [v * 4 + iota // 4])
        gb_ref[pl.ds(pl.multiple_of(v * 16, 16), 16)] = \
            plsc.load_gather(bx_ref, [qv * 4 + iota % 4])
        return 0

    lax.fori_loop(0, k_pad * 4 // 16, gbvec, 0)

    fl_ref[...] = jnp.full((16,), flag, jnp.int32)
    pltpu.sync_copy(sc_ref, scores_hbm.at[pl.ds(b * k_pad, k_pad)])
    pltpu.sync_copy(la_ref, labels_hbm.at[pl.ds(b * k_pad, k_pad)])
    pltpu.sync_copy(gb_ref, boxout_hbm.at[pl.ds(b * k_pad * 4, k_pad * 4)])
    pltpu.sync_copy(fl_ref, flag_hbm.at[pl.ds(b * 16, 16)])


def _slow_kernel(x_ref, box_ref, scores_ref, labels_ref, boxes_ref, p_ref, m_ref,
                 *, Q, C, n_chunks, n_select):
    for i in range(n_chunks):
        lo = i * 128
        hi = min(lo + 128, Q)
        rows = hi - lo
        p_ref[i, 0:rows, :] = jax.nn.sigmoid(x_ref[0, lo:hi, :])
        if rows < 128:
            p_ref[i, rows:128, :] = jnp.full((128 - rows, C), -1.0, jnp.float32)

    m_ref[...] = jnp.max(p_ref[...], axis=2)

    iota_flat = jax.lax.broadcasted_iota(jnp.int32, (n_chunks, 128), 0) * 128 + \
        jax.lax.broadcasted_iota(jnp.int32, (n_chunks, 128), 1)
    iota_c = jax.lax.broadcasted_iota(jnp.int32, (1, 1, C), 2)
    iota4 = jax.lax.broadcasted_iota(jnp.int32, (1, 4), 1)

    def body(it, carry):
        mv = m_ref[...]
        m = jnp.max(mv)
        q = jnp.min(jnp.where(mv == m, iota_flat, _BIG))
        ci = q // 128
        sj = q - ci * 128
        row = p_ref[pl.ds(ci, 1), pl.ds(sj, 1), :]
        c = jnp.min(jnp.where(row == m, iota_c, _BIG))
        p_ref[pl.ds(ci, 1), pl.ds(sj, 1), :] = jnp.where(iota_c == c, -1.0, row)
        m2 = jnp.max(jnp.where(iota_c == c, -1.0, row))
        m_ref[...] = jnp.where(iota_flat == q, m2, mv)

        scores_ref[0, pl.ds(it, 1), :] = jnp.full((1, 1), m, jnp.float32)
        labels_ref[0, pl.ds(it, 1), :] = jnp.full((1, 1), c, jnp.int32)

        bx = box_ref[0, pl.ds(q, 1), :]
        cx = jnp.sum(jnp.where(iota4 == 0, bx, 0.0))
        cy = jnp.sum(jnp.where(iota4 == 1, bx, 0.0))
        w = jnp.sum(jnp.where(iota4 == 2, bx, 0.0))
        h = jnp.sum(jnp.where(iota4 == 3, bx, 0.0))
        xyxy = jnp.where(iota4 == 0, cx - 0.5 * w,
                         jnp.where(iota4 == 1, cy - 0.5 * h,
                                   jnp.where(iota4 == 2, cx + 0.5 * w, cy + 0.5 * h)))
        boxes_ref[0, pl.ds(it, 1), :] = xyxy
        return carry

    jax.lax.fori_loop(0, n_select, body, 0)


def _slow_extract(pred_logits, pred_boxes):
    B, Q, C = pred_logits.shape
    n_chunks = (Q + 127) // 128
    k = _NUM_SELECT
    k_pad = ((k + 7) // 8) * 8
    kfun = functools.partial(_slow_kernel, Q=Q, C=C, n_chunks=n_chunks, n_select=k)
    scores3, labels3, boxes3 = pl.pallas_call(
        kfun,
        grid=(B,),
        in_specs=[
            pl.BlockSpec((1, Q, C), lambda b: (b, 0, 0)),
            pl.BlockSpec((1, Q, 4), lambda b: (b, 0, 0)),
        ],
        out_specs=[
            pl.BlockSpec((1, k_pad, 1), lambda b: (b, 0, 0)),
            pl.BlockSpec((1, k_pad, 1), lambda b: (b, 0, 0)),
            pl.BlockSpec((1, k_pad, 4), lambda b: (b, 0, 0)),
        ],
        out_shape=[
            jax.ShapeDtypeStruct((B, k_pad, 1), jnp.float32),
            jax.ShapeDtypeStruct((B, k_pad, 1), jnp.int32),
            jax.ShapeDtypeStruct((B, k_pad, 4), jnp.float32),
        ],
        scratch_shapes=[
            pltpu.VMEM((n_chunks, 128, C), jnp.float32),
            pltpu.VMEM((n_chunks, 128), jnp.float32),
        ],
    )(pred_logits, pred_boxes)
    return scores3[:, :k, 0], labels3[:, :k, 0], boxes3[:, :k, :]


def kernel(pred_logits, pred_boxes, target_sizes):
    B, Q, C = pred_logits.shape
    n_chunks = (Q + 127) // 128
    n_rows = n_chunks * 128
    k = _NUM_SELECT
    k_pad = ((k + 15) // 16) * 16

    lfun = functools.partial(_lists_kernel, Q=Q, C=C, n_chunks=n_chunks)
    vlists, clists = pl.pallas_call(
        lfun,
        grid=(B,),
        in_specs=[pl.BlockSpec((1, Q, C), lambda b: (b, 0, 0))],
        out_specs=[
            pl.BlockSpec((1, _DEPTH, n_chunks, 128), lambda b: (b, 0, 0, 0)),
            pl.BlockSpec((1, _DEPTH, n_chunks, 128), lambda b: (b, 0, 0, 0)),
        ],
        out_shape=[
            jax.ShapeDtypeStruct((B, _DEPTH, n_chunks, 128), jnp.float32),
            jax.ShapeDtypeStruct((B, _DEPTH, n_chunks, 128), jnp.int32),
        ],
        scratch_shapes=[pltpu.VMEM((n_chunks, 128, C), jnp.float32)],
    )(pred_logits)

    mesh = plsc.VectorSubcoreMesh(core_axis_name="c", subcore_axis_name="s")
    sfun = functools.partial(_sc_extract, Q=Q, n_rows=n_rows, k_pad=k_pad,
                             n_select=k)
    sc_call = pl.kernel(
        sfun,
        mesh=mesh,
        compiler_params=pltpu.CompilerParams(needs_layout_passes=False),
        out_type=[
            jax.ShapeDtypeStruct((B * k_pad,), jnp.float32),
            jax.ShapeDtypeStruct((B * k_pad,), jnp.int32),
            jax.ShapeDtypeStruct((B * k_pad * 4,), jnp.float32),
            jax.ShapeDtypeStruct((B * 16,), jnp.int32),
        ],
        scratch_types=[
            pltpu.VMEM((_DEPTH * n_rows,), jnp.float32),
            pltpu.VMEM((_DEPTH * n_rows,), jnp.int32),
            pltpu.VMEM((Q * 4,), jnp.float32),
            pltpu.VMEM((n_rows,), jnp.float32),
            pltpu.VMEM((n_rows // 16,), jnp.float32),
            pltpu.VMEM((n_rows,), jnp.int32),
            pltpu.VMEM((k_pad,), jnp.float32),
            pltpu.VMEM((k_pad,), jnp.int32),
            pltpu.VMEM((k_pad,), jnp.int32),
            pltpu.VMEM((k_pad * 4,), jnp.float32),
            pltpu.VMEM((16,), jnp.int32),
        ],
    )
    scores_f, labels_f, gbox, flag = sc_call(
        vlists.reshape(B * _DEPTH * n_rows),
        clists.reshape(B * _DEPTH * n_rows),
        pred_boxes.reshape(B * Q * 4),
    )
    scores_f = scores_f.reshape(B, k_pad)
    labels_f = labels_f.reshape(B, k_pad)
    gbox = gbox.reshape(B, k_pad, 4)

    def _fast():
        gb = gbox[:, :k, :]
        cx, cy, w, h = (gb[..., i] for i in range(4))
        bx = jnp.stack([cx - 0.5 * w, cy - 0.5 * h, cx + 0.5 * w, cy + 0.5 * h],
                       axis=-1)
        return scores_f[:, :k], labels_f[:, :k], bx

    use_slow = jnp.any(flag != 0)
    scores, labels, boxes = lax.cond(
        use_slow, lambda: _slow_extract(pred_logits, pred_boxes), _fast)

    img_h = target_sizes[:, 0]
    img_w = target_sizes[:, 1]
    scale_fct = jnp.stack([img_w, img_h, img_w, img_h], axis=1).astype(jnp.float32)
    return scores, labels, boxes * scale_fct[:, None, :]


# speedup vs baseline: 51.0414x; 1.3616x over previous
import functools

import jax
import jax.numpy as jnp
from jax import lax
from jax.experimental import pallas as pl
from jax.experimental.pallas import tpu as pltpu
from jax.experimental.pallas import tpu_sc as plsc

_NUM_SELECT = 300
_BIG = 2**30
_NEG = -2.0


def _lists_kernel(x_ref, p_ref, v_ref, c_ref, *, Q, C, n_chunks):
    for i in range(n_chunks):
        lo = i * 128
        hi = min(lo + 128, Q)
        rows = hi - lo
        p_ref[0, i, 0:rows, :] = jax.nn.sigmoid(x_ref[0, lo:hi, :])
        if rows < 128:
            p_ref[0, i, rows:128, :] = jnp.full((128 - rows, C), -1.0, jnp.float32)

    iota_c3 = jax.lax.broadcasted_iota(jnp.int32, (n_chunks, 128, C), 2)
    p = p_ref[0]
    v0 = jnp.max(p, axis=2)
    c0 = jnp.min(jnp.where(p == v0[..., None], iota_c3, _BIG), axis=2)
    valid = (p < v0[..., None]) | ((p == v0[..., None]) & (iota_c3 > c0[..., None]))
    pm = jnp.where(valid, p, -1.0)
    v1 = jnp.max(pm, axis=2)
    c1 = jnp.min(jnp.where(pm == v1[..., None], iota_c3, _BIG), axis=2)
    v_ref[0, 0] = v0
    v_ref[0, 1] = v1
    c_ref[0, 0] = c0
    c_ref[0, 1] = c1


def _sc_extract(v_hbm, c_hbm, p_hbm, boxes_hbm, scores_hbm, labels_hbm,
                boxout_hbm, vv, cv, row_ref, bx_ref, m_ref, m2_ref, k_ref,
                cc_ref, sc_ref, la_ref, bi_ref, gb_ref,
                *, Q, C, n_rows, k_pad, n_select):
    b = lax.axis_index("s") * 2 + lax.axis_index("c")
    nl = 2 * n_rows
    pltpu.sync_copy(v_hbm.at[pl.ds(b * nl, nl)], vv)
    pltpu.sync_copy(c_hbm.at[pl.ds(b * nl, nl)], cv)
    pltpu.sync_copy(boxes_hbm.at[pl.ds(b * Q * 4, Q * 4)], bx_ref)
    iota = lax.broadcasted_iota(jnp.int32, (16,), 0)
    n_vec = n_rows // 16
    n_vec2 = n_vec // 16
    n_cvec = C // 16

    def init_m(g, _):
        sl = pl.ds(pl.multiple_of(g * 16, 16), 16)
        vvec = vv[sl]
        m_ref[sl] = vvec
        cc_ref[sl] = cv[sl]
        k_ref[sl] = jnp.zeros((16,), jnp.int32)
        base = (g // 16) * 16
        sl2 = pl.ds(pl.multiple_of(base, 16), 16)
        m2_ref[sl2] = jnp.where(iota == g - base, jnp.max(vvec), m2_ref[sl2])
        return 0

    lax.fori_loop(0, n_vec, init_m, 0)

    def init_out(j, _):
        sl = pl.ds(pl.multiple_of(j * 16, 16), 16)
        sc_ref[sl] = jnp.zeros((16,), jnp.float32)
        la_ref[sl] = jnp.zeros((16,), jnp.int32)
        bi_ref[sl] = jnp.zeros((16,), jnp.int32)
        return 0

    lax.fori_loop(0, k_pad // 16, init_out, 0)

    def step(it, carry):
        def mx(i, acc):
            return jnp.maximum(acc, jnp.max(m2_ref[pl.ds(pl.multiple_of(i * 16, 16), 16)]))
        m = lax.fori_loop(0, n_vec2, mx, jnp.float32(-3.0))

        def fe(i, acc):
            v2 = m2_ref[pl.ds(pl.multiple_of(i * 16, 16), 16)]
            return jnp.minimum(acc, jnp.min(jnp.where(v2 == m, iota + i * 16, _BIG)))
        g = lax.fori_loop(0, n_vec2, fe, jnp.int32(_BIG))

        gs = pl.multiple_of(g * 16, 16)
        mv = m_ref[pl.ds(gs, 16)]
        q = jnp.min(jnp.where(mv == m, iota + g * 16, _BIG))
        lane_q = q - g * 16
        kvec = k_ref[pl.ds(gs, 16)]
        kq = jnp.max(jnp.where(iota == lane_q, kvec, 0))
        cvec = cc_ref[pl.ds(gs, 16)]
        cval = jnp.max(jnp.where(iota == lane_q, cvec, 0))

        def _fast_next():
            nvec = vv[pl.ds(pl.multiple_of(n_rows + gs, 16), 16)]
            ncvec = cv[pl.ds(pl.multiple_of(n_rows + gs, 16), 16)]
            nv = jnp.max(jnp.where(iota == lane_q, nvec, jnp.float32(_NEG)))
            nc = jnp.max(jnp.where(iota == lane_q, ncvec, 0))
            return nv, nc

        def _refill():
            off = (b * n_rows + q) * C
            pltpu.sync_copy(p_hbm.at[pl.ds(off, C)], row_ref)

            def mx2(j, acc):
                pv = row_ref[pl.ds(pl.multiple_of(j * 16, 16), 16)]
                cvi = iota + j * 16
                ok = (pv < m) | ((pv == m) & (cvi > cval))
                return jnp.maximum(acc, jnp.max(jnp.where(ok, pv, jnp.float32(_NEG))))

            nv = lax.fori_loop(0, n_cvec, mx2, jnp.float32(_NEG))

            def mc2(j, acc):
                pv = row_ref[pl.ds(pl.multiple_of(j * 16, 16), 16)]
                cvi = iota + j * 16
                ok = (pv < m) | ((pv == m) & (cvi > cval))
                return jnp.minimum(acc, jnp.min(jnp.where(ok & (pv == nv), cvi, _BIG)))

            nc = lax.fori_loop(0, n_cvec, mc2, jnp.int32(_BIG))
            return nv, nc

        nval, ncl = lax.cond(kq == 0, _fast_next, _refill)

        mv2 = jnp.where(iota == lane_q, nval, mv)
        m_ref[pl.ds(gs, 16)] = mv2
        base = (g // 16) * 16
        sl2 = pl.ds(pl.multiple_of(base, 16), 16)
        m2_ref[sl2] = jnp.where(iota == g - base, jnp.max(mv2), m2_ref[sl2])
        k_ref[pl.ds(gs, 16)] = jnp.where(iota == lane_q, kq + 1, kvec)
        cc_ref[pl.ds(gs, 16)] = jnp.where(iota == lane_q, ncl, cvec)

        ob = (it // 16) * 16
        slo = pl.ds(pl.multiple_of(ob, 16), 16)
        lane_it = it - ob
        sc_ref[slo] = jnp.where(iota == lane_it, m, sc_ref[slo])
        la_ref[slo] = jnp.where(iota == lane_it, cval, la_ref[slo])
        bi_ref[slo] = jnp.where(iota == lane_it, q, bi_ref[slo])
        return 0

    lax.fori_loop(0, n_select, step, 0)

    def gbvec(v, _):
        qv = plsc.load_gather(bi_ref, [v * 4 + iota // 4])
        gb_ref[pl.ds(pl.multiple_of(v * 16, 16), 16)] = \
            plsc.load_gather(bx_ref, [qv * 4 + iota % 4])
        return 0

    lax.fori_loop(0, k_pad * 4 // 16, gbvec, 0)

    pltpu.sync_copy(sc_ref, scores_hbm.at[pl.ds(b * k_pad, k_pad)])
    pltpu.sync_copy(la_ref, labels_hbm.at[pl.ds(b * k_pad, k_pad)])
    pltpu.sync_copy(gb_ref, boxout_hbm.at[pl.ds(b * k_pad * 4, k_pad * 4)])


def kernel(pred_logits, pred_boxes, target_sizes):
    B, Q, C = pred_logits.shape
    n_chunks = (Q + 127) // 128
    n_rows = n_chunks * 128
    k = _NUM_SELECT
    k_pad = ((k + 15) // 16) * 16

    lfun = functools.partial(_lists_kernel, Q=Q, C=C, n_chunks=n_chunks)
    probs, vlists, clists = pl.pallas_call(
        lfun,
        grid=(B,),
        in_specs=[pl.BlockSpec((1, Q, C), lambda b: (b, 0, 0))],
        out_specs=[
            pl.BlockSpec((1, n_chunks, 128, C), lambda b: (b, 0, 0, 0)),
            pl.BlockSpec((1, 2, n_chunks, 128), lambda b: (b, 0, 0, 0)),
            pl.BlockSpec((1, 2, n_chunks, 128), lambda b: (b, 0, 0, 0)),
        ],
        out_shape=[
            jax.ShapeDtypeStruct((B, n_chunks, 128, C), jnp.float32),
            jax.ShapeDtypeStruct((B, 2, n_chunks, 128), jnp.float32),
            jax.ShapeDtypeStruct((B, 2, n_chunks, 128), jnp.int32),
        ],
    )(pred_logits)

    mesh = plsc.VectorSubcoreMesh(core_axis_name="c", subcore_axis_name="s")
    sfun = functools.partial(_sc_extract, Q=Q, C=C, n_rows=n_rows, k_pad=k_pad,
                             n_select=k)
    sc_call = pl.kernel(
        sfun,
        mesh=mesh,
        compiler_params=pltpu.CompilerParams(needs_layout_passes=False),
        out_type=[
            jax.ShapeDtypeStruct((B * k_pad,), jnp.float32),
            jax.ShapeDtypeStruct((B * k_pad,), jnp.int32),
            jax.ShapeDtypeStruct((B * k_pad * 4,), jnp.float32),
        ],
        scratch_types=[
            pltpu.VMEM((2 * n_rows,), jnp.float32),
            pltpu.VMEM((2 * n_rows,), jnp.int32),
            pltpu.VMEM((C,), jnp.float32),
            pltpu.VMEM((Q * 4,), jnp.float32),
            pltpu.VMEM((n_rows,), jnp.float32),
            pltpu.VMEM((n_rows // 16,), jnp.float32),
            pltpu.VMEM((n_rows,), jnp.int32),
            pltpu.VMEM((n_rows,), jnp.int32),
            pltpu.VMEM((k_pad,), jnp.float32),
            pltpu.VMEM((k_pad,), jnp.int32),
            pltpu.VMEM((k_pad,), jnp.int32),
            pltpu.VMEM((k_pad * 4,), jnp.float32),
        ],
    )
    scores_f, labels_f, gbox = sc_call(
        vlists.reshape(B * 2 * n_rows),
        clists.reshape(B * 2 * n_rows),
        probs.reshape(B * n_rows * C),
        pred_boxes.reshape(B * Q * 4),
    )
    scores_f = scores_f.reshape(B, k_pad)
    labels_f = labels_f.reshape(B, k_pad)
    gbox = gbox.reshape(B, k_pad, 4)

    gb = gbox[:, :k, :]
    cx, cy, w, h = (gb[..., i] for i in range(4))
    boxes = jnp.stack([cx - 0.5 * w, cy - 0.5 * h, cx + 0.5 * w, cy + 0.5 * h],
                      axis=-1)
    scores = scores_f[:, :k]
    labels = labels_f[:, :k]

    img_h = target_sizes[:, 0]
    img_w = target_sizes[:, 1]
    scale_fct = jnp.stack([img_w, img_h, img_w, img_h], axis=1).astype(jnp.float32)
    return scores, labels, boxes * scale_fct[:, None, :]
